# baseline (speedup 1.0000x reference)
import functools

import jax
import jax.numpy as jnp
from jax import lax
from jax.experimental import pallas as pl
from jax.experimental.pallas import tpu as pltpu
from jax.experimental.pallas import tpu_sc as plsc

NC = 2
NS = 16
L = 16
NW = NC * NS

NPW = 320
CH = 8000
RB = 256
SB = RB // L
U = 5
SHIFT = 14
MASKV = (1 << SHIFT) - 1


def _pack_body(ei_ref, pk_ref):
    src = ei_ref[0]
    dst = ei_ref[1]
    pk_ref[...] = dst * (1 << SHIFT) + src


def _mm_body(x_ref, w_ref, b_ref, a_ref, bm_ref):
    xb = x_ref[...]
    d = x_ref.shape[1]
    w1m2 = w_ref[:, :d] - w_ref[:, d:]
    w2 = w_ref[:, d:]
    dn = (((1,), (1,)), ((), ()))
    a_ref[...] = (
        lax.dot_general(xb, w1m2, dn, preferred_element_type=jnp.float32)
        + b_ref[...]
    )
    bm_ref[...] = lax.dot_general(xb, w2, dn, preferred_element_type=jnp.float32)


def _segmax_body(nchunks, bm_hbm, pk_hbm, out_hbm,
                 acc_v, pkc0, pkc1, mdl0, mdl1, msrc0, msrc1,
                 rows_v, gsem, csem0, csem1):
    cid = lax.axis_index("c")
    sid = lax.axis_index("s")
    wid = sid * NC + cid
    lo = wid * NPW
    hi = lo + NPW
    neg_inf = jnp.full((L,), -jnp.inf, dtype=jnp.float32)
    d = bm_hbm.shape[1]
    dv_per_row = d // L

    def _init(i, carry):
        r = i // dv_per_row
        v = i - r * dv_per_row
        acc_v[r, pl.ds(v * L, L)] = neg_inf
        return carry

    lax.fori_loop(0, (NPW + 1) * dv_per_row, _init, 0)

    def scan_chunk(pkc, mdl, msrc):
        def scan_body(k, m):
            base = k * (L * U)
            for u in range(U):
                pk = pkc[pl.ds(base + u * L, L)]
                dv = lax.shift_right_logical(pk, SHIFT)
                mask = (dv >= lo) & (dv < hi)
                plsc.store_compressed(mdl.at[pl.ds(m, L)], dv - lo, mask=mask)
                plsc.store_compressed(msrc.at[pl.ds(m, L)], pk & MASKV, mask=mask)
                m = m + plsc.all_reduce_population_count(mask)[0]
            return m

        m = lax.fori_loop(0, CH // (L * U), scan_body, jnp.int32(0))
        padidx = m + lax.iota(jnp.int32, L)
        plsc.store_scatter(mdl, [padidx], jnp.full((L,), NPW, jnp.int32))
        plsc.store_scatter(msrc, [padidx], jnp.zeros((L,), jnp.int32))
        return m

    def fire_gathers(msrc, j0, nj):
        def fire(jj, c2):
            pltpu.async_copy(
                bm_hbm.at[msrc.at[pl.ds((j0 + jj) * L, L)]],
                rows_v.at[pl.ds(jj * L, L)], gsem)
            return c2
        lax.fori_loop(0, nj, fire, 0)

    def drain_gathers(msrc, j0, nj):
        def drain(jj, c2):
            pltpu.make_async_copy(
                bm_hbm.at[msrc.at[pl.ds((j0 + jj) * L, L)]],
                rows_v.at[pl.ds(jj * L, L)], gsem).wait()
            return c2
        lax.fori_loop(0, nj, drain, 0)

    def accumulate(mdl, j0, nj):
        def edge_body(e, c2):
            dlv = mdl[pl.ds(j0 * L + e, L)]
            dl = dlv[0]
            for v in range(dv_per_row):
                sl = pl.ds(v * L, L)
                acc_v[dl, sl] = jnp.maximum(acc_v[dl, sl], rows_v[e, sl])
            return c2
        lax.fori_loop(0, nj * L, edge_body, 0)

    def process_matches(mdl, msrc, m, first_fired):
        jmax = (m + L - 1) // L

        def wave_body(wv, c2):
            j0 = wv * SB
            nj = jnp.minimum(jmax - j0, SB)
            if not first_fired:
                fire_gathers(msrc, j0, nj)
            else:
                @pl.when(wv > 0)
                def _():
                    fire_gathers(msrc, j0, nj)

            def sub_body(jj, c3):
                pltpu.make_async_copy(
                    bm_hbm.at[msrc.at[pl.ds((j0 + jj) * L, L)]],
                    rows_v.at[pl.ds(jj * L, L)], gsem).wait()

                def edge_body(e, c4):
                    dlv = mdl[pl.ds((j0 + jj) * L + e, L)]
                    dl = dlv[0]
                    for v in range(dv_per_row):
                        sl = pl.ds(v * L, L)
                        acc_v[dl, sl] = jnp.maximum(
                            acc_v[dl, sl], rows_v[jj * L + e, sl])
                    return c4

                lax.fori_loop(0, L, edge_body, 0)
                return c3

            lax.fori_loop(0, nj, sub_body, 0)
            return c2

        nwaves = (jmax + SB - 1) // SB
        lax.fori_loop(0, nwaves, wave_body, 0)

    def load_chunk_start(g, pkc, csem):
        pltpu.async_copy(pk_hbm.at[pl.ds(g * CH, CH)], pkc, csem)

    def load_chunk_wait(g, pkc, csem):
        pltpu.make_async_copy(pk_hbm.at[pl.ds(g * CH, CH)], pkc, csem).wait()

    load_chunk_start(0, pkc0, csem0)
    load_chunk_start(1, pkc1, csem1)
    load_chunk_wait(0, pkc0, csem0)
    m0 = scan_chunk(pkc0, mdl0, msrc0)

    def pair_body(gg, mP):
        g = 2 * gg + 1
        nj0 = jnp.minimum((mP + L - 1) // L, SB)
        fire_gathers(msrc0, 0, nj0)
        load_chunk_wait(g, pkc1, csem1)

        @pl.when(g + 1 < nchunks)
        def _():
            load_chunk_start(g + 1, pkc0, csem0)

        m1 = scan_chunk(pkc1, mdl1, msrc1)
        process_matches(mdl0, msrc0, mP, first_fired=True)

        g2 = g + 1

        def do_even():
            nj1 = jnp.minimum((m1 + L - 1) // L, SB)
            fire_gathers(msrc1, 0, nj1)
            load_chunk_wait(g2, pkc0, csem0)

            @pl.when(g2 + 1 < nchunks)
            def _():
                load_chunk_start(g2 + 1, pkc1, csem1)

            m2 = scan_chunk(pkc0, mdl0, msrc0)
            process_matches(mdl1, msrc1, m1, first_fired=True)
            return m2

        return lax.cond(g2 < nchunks, do_even, lambda: m1)

    mF = lax.fori_loop(0, nchunks // 2, pair_body, m0)
    process_matches(mdl1, msrc1, mF, first_fired=False)

    pltpu.sync_copy(acc_v.at[pl.ds(0, NPW)], out_hbm.at[pl.ds(lo, NPW)])


def _combine_body(a_ref, seg_ref, h_ref, sum_ref, sumsq_ref):
    agg = a_ref[...] + seg_ref[...]
    agg = jnp.where(jnp.isfinite(agg), agg, 0.0)
    h = jnp.where(agg > 0, agg, jnp.exp(jnp.minimum(agg, 0.0)) - 1.0)
    h_ref[...] = h
    sum_ref[...] = jnp.sum(h, axis=0, keepdims=True)
    sumsq_ref[...] = jnp.sum(h * h, axis=0, keepdims=True)


def _norm_body(h_ref, sum_ref, sumsq_ref, gamma_ref, beta_ref, out_ref):
    n = h_ref.shape[0]
    mean = sum_ref[...] / n
    var = sumsq_ref[...] / n - mean * mean
    inv = lax.rsqrt(var + 1e-5)
    out_ref[...] = gamma_ref[...] * (h_ref[...] - mean) * inv + beta_ref[...]


def kernel(x, edge_index, W, b, gamma, beta):
    n, d = x.shape
    e = edge_index.shape[1]
    npad = NW * NPW
    assert e % CH == 0 and (e // CH) % 2 == 0 and CH % (L * U) == 0

    ei3 = edge_index.reshape(2, e // 128, 128)
    pk2d = pl.pallas_call(
        _pack_body,
        out_shape=jax.ShapeDtypeStruct((e // 128, 128), jnp.int32),
    )(ei3)
    pk = pk2d.reshape(e)

    a_mat, bm = pl.pallas_call(
        _mm_body,
        out_shape=[
            jax.ShapeDtypeStruct((n, d), jnp.float32),
            jax.ShapeDtypeStruct((n, d), jnp.float32),
        ],
    )(x, W, b.reshape(1, d))

    mesh = plsc.VectorSubcoreMesh(core_axis_name="c", subcore_axis_name="s")
    seg_full = pl.kernel(
        functools.partial(_segmax_body, e // CH),
        out_type=jax.ShapeDtypeStruct((npad, d), jnp.float32),
        mesh=mesh,
        scratch_types=[
            pltpu.VMEM((NPW + 1, d), jnp.float32),
            pltpu.VMEM((CH,), jnp.int32),
            pltpu.VMEM((CH,), jnp.int32),
            pltpu.VMEM((CH + 2 * L,), jnp.int32),
            pltpu.VMEM((CH + 2 * L,), jnp.int32),
            pltpu.VMEM((CH + 2 * L,), jnp.int32),
            pltpu.VMEM((CH + 2 * L,), jnp.int32),
            pltpu.VMEM((RB, d), jnp.float32),
            pltpu.SemaphoreType.DMA,
            pltpu.SemaphoreType.DMA,
            pltpu.SemaphoreType.DMA,
        ],
        compiler_params=pltpu.CompilerParams(needs_layout_passes=False),
    )(bm, pk)
    seg = seg_full[:n]

    h, s1, s2 = pl.pallas_call(
        _combine_body,
        out_shape=[
            jax.ShapeDtypeStruct((n, d), jnp.float32),
            jax.ShapeDtypeStruct((1, d), jnp.float32),
            jax.ShapeDtypeStruct((1, d), jnp.float32),
        ],
    )(a_mat, seg)
    out = pl.pallas_call(
        _norm_body,
        out_shape=jax.ShapeDtypeStruct((n, d), jnp.float32),
    )(h, s1, s2, gamma.reshape(1, d), beta.reshape(1, d))
    return out


# speedup vs baseline: 1.0084x; 1.0084x over previous
import functools

import jax
import jax.numpy as jnp
from jax import lax
from jax.experimental import pallas as pl
from jax.experimental.pallas import tpu as pltpu
from jax.experimental.pallas import tpu_sc as plsc

NC = 2
NS = 16
L = 16
NW = NC * NS

NPW = 320
CH = 6400
RB = 256
SB = RB // L
U = 8
SHIFT = 14
MASKV = (1 << SHIFT) - 1


def _pack_body(ei_ref, pk_ref):
    src = ei_ref[0]
    dst = ei_ref[1]
    pk_ref[...] = dst * (1 << SHIFT) + src


def _mm_body(x_ref, w_ref, b_ref, a_ref, bm_ref):
    xb = x_ref[...]
    d = x_ref.shape[1]
    w1m2 = w_ref[:, :d] - w_ref[:, d:]
    w2 = w_ref[:, d:]
    dn = (((1,), (1,)), ((), ()))
    a_ref[...] = (
        lax.dot_general(xb, w1m2, dn, preferred_element_type=jnp.float32)
        + b_ref[...]
    )
    bm_ref[...] = lax.dot_general(xb, w2, dn, preferred_element_type=jnp.float32)


def _segmax_body(nchunks, bm_hbm, pk_hbm, out_hbm,
                 acc_v, pkc0, pkc1, mdl0, mdl1, msrc0, msrc1,
                 rows_v, gsem, csem0, csem1):
    cid = lax.axis_index("c")
    sid = lax.axis_index("s")
    wid = sid * NC + cid
    lo = wid * NPW
    hi = lo + NPW
    neg_inf = jnp.full((L,), -jnp.inf, dtype=jnp.float32)
    d = bm_hbm.shape[1]
    dv_per_row = d // L

    def _init(i, carry):
        r = i // dv_per_row
        v = i - r * dv_per_row
        acc_v[r, pl.ds(v * L, L)] = neg_inf
        return carry

    lax.fori_loop(0, (NPW + 1) * dv_per_row, _init, 0)

    def scan_chunk(pkc, mdl, msrc):
        def scan_body(k, m):
            base = k * (L * U)
            for u in range(U):
                pk = pkc[pl.ds(base + u * L, L)]
                dv = lax.shift_right_logical(pk, SHIFT)
                mask = (dv >= lo) & (dv < hi)
                plsc.store_compressed(mdl.at[pl.ds(m, L)], dv - lo, mask=mask)
                plsc.store_compressed(msrc.at[pl.ds(m, L)], pk & MASKV, mask=mask)
                m = m + plsc.all_reduce_population_count(mask)[0]
            return m

        m = lax.fori_loop(0, CH // (L * U), scan_body, jnp.int32(0))
        padidx = m + lax.iota(jnp.int32, L)
        plsc.store_scatter(mdl, [padidx], jnp.full((L,), NPW, jnp.int32))
        plsc.store_scatter(msrc, [padidx], jnp.zeros((L,), jnp.int32))
        return m

    def fire_gathers(msrc, j0, nj):
        def fire(jj, c2):
            pltpu.async_copy(
                bm_hbm.at[msrc.at[pl.ds((j0 + jj) * L, L)]],
                rows_v.at[pl.ds(jj * L, L)], gsem)
            return c2
        lax.fori_loop(0, nj, fire, 0)

    def drain_gathers(msrc, j0, nj):
        def drain(jj, c2):
            pltpu.make_async_copy(
                bm_hbm.at[msrc.at[pl.ds((j0 + jj) * L, L)]],
                rows_v.at[pl.ds(jj * L, L)], gsem).wait()
            return c2
        lax.fori_loop(0, nj, drain, 0)

    def accumulate(mdl, j0, nj):
        def edge_body(e, c2):
            dlv = mdl[pl.ds(j0 * L + e, L)]
            dl = dlv[0]
            for v in range(dv_per_row):
                sl = pl.ds(v * L, L)
                acc_v[dl, sl] = jnp.maximum(acc_v[dl, sl], rows_v[e, sl])
            return c2
        lax.fori_loop(0, nj * L, edge_body, 0)

    def process_matches(mdl, msrc, m, first_fired):
        jmax = (m + L - 1) // L

        def wave_body(wv, c2):
            j0 = wv * SB
            nj = jnp.minimum(jmax - j0, SB)
            if not first_fired:
                fire_gathers(msrc, j0, nj)
            else:
                @pl.when(wv > 0)
                def _():
                    fire_gathers(msrc, j0, nj)
            drain_gathers(msrc, j0, nj)
            accumulate(mdl, j0, nj)
            return c2

        nwaves = (jmax + SB - 1) // SB
        lax.fori_loop(0, nwaves, wave_body, 0)

    def load_chunk_start(g, pkc, csem):
        pltpu.async_copy(pk_hbm.at[pl.ds(g * CH, CH)], pkc, csem)

    def load_chunk_wait(g, pkc, csem):
        pltpu.make_async_copy(pk_hbm.at[pl.ds(g * CH, CH)], pkc, csem).wait()

    load_chunk_start(0, pkc0, csem0)
    load_chunk_start(1, pkc1, csem1)
    load_chunk_wait(0, pkc0, csem0)
    m0 = scan_chunk(pkc0, mdl0, msrc0)

    def pair_body(gg, mP):
        g = 2 * gg + 1
        nj0 = jnp.minimum((mP + L - 1) // L, SB)
        fire_gathers(msrc0, 0, nj0)
        load_chunk_wait(g, pkc1, csem1)

        @pl.when(g + 1 < nchunks)
        def _():
            load_chunk_start(g + 1, pkc0, csem0)

        m1 = scan_chunk(pkc1, mdl1, msrc1)
        process_matches(mdl0, msrc0, mP, first_fired=True)

        g2 = g + 1

        def do_even():
            nj1 = jnp.minimum((m1 + L - 1) // L, SB)
            fire_gathers(msrc1, 0, nj1)
            load_chunk_wait(g2, pkc0, csem0)

            @pl.when(g2 + 1 < nchunks)
            def _():
                load_chunk_start(g2 + 1, pkc1, csem1)

            m2 = scan_chunk(pkc0, mdl0, msrc0)
            process_matches(mdl1, msrc1, m1, first_fired=True)
            return m2

        return lax.cond(g2 < nchunks, do_even, lambda: m1)

    mF = lax.fori_loop(0, nchunks // 2, pair_body, m0)
    process_matches(mdl1, msrc1, mF, first_fired=False)

    pltpu.sync_copy(acc_v.at[pl.ds(0, NPW)], out_hbm.at[pl.ds(lo, NPW)])


def _combine_body(a_ref, seg_ref, h_ref, sum_ref, sumsq_ref):
    agg = a_ref[...] + seg_ref[...]
    agg = jnp.where(jnp.isfinite(agg), agg, 0.0)
    h = jnp.where(agg > 0, agg, jnp.exp(jnp.minimum(agg, 0.0)) - 1.0)
    h_ref[...] = h
    sum_ref[...] = jnp.sum(h, axis=0, keepdims=True)
    sumsq_ref[...] = jnp.sum(h * h, axis=0, keepdims=True)


def _norm_body(h_ref, sum_ref, sumsq_ref, gamma_ref, beta_ref, out_ref):
    n = h_ref.shape[0]
    mean = sum_ref[...] / n
    var = sumsq_ref[...] / n - mean * mean
    inv = lax.rsqrt(var + 1e-5)
    out_ref[...] = gamma_ref[...] * (h_ref[...] - mean) * inv + beta_ref[...]


def kernel(x, edge_index, W, b, gamma, beta):
    n, d = x.shape
    e = edge_index.shape[1]
    npad = NW * NPW
    assert e % CH == 0 and (e // CH) % 2 == 0 and CH % (L * U) == 0

    ei3 = edge_index.reshape(2, e // 128, 128)
    pk2d = pl.pallas_call(
        _pack_body,
        out_shape=jax.ShapeDtypeStruct((e // 128, 128), jnp.int32),
    )(ei3)
    pk = pk2d.reshape(e)

    a_mat, bm = pl.pallas_call(
        _mm_body,
        out_shape=[
            jax.ShapeDtypeStruct((n, d), jnp.float32),
            jax.ShapeDtypeStruct((n, d), jnp.float32),
        ],
    )(x, W, b.reshape(1, d))

    mesh = plsc.VectorSubcoreMesh(core_axis_name="c", subcore_axis_name="s")
    seg_full = pl.kernel(
        functools.partial(_segmax_body, e // CH),
        out_type=jax.ShapeDtypeStruct((npad, d), jnp.float32),
        mesh=mesh,
        scratch_types=[
            pltpu.VMEM((NPW + 1, d), jnp.float32),
            pltpu.VMEM((CH,), jnp.int32),
            pltpu.VMEM((CH,), jnp.int32),
            pltpu.VMEM((CH + 2 * L,), jnp.int32),
            pltpu.VMEM((CH + 2 * L,), jnp.int32),
            pltpu.VMEM((CH + 2 * L,), jnp.int32),
            pltpu.VMEM((CH + 2 * L,), jnp.int32),
            pltpu.VMEM((RB, d), jnp.float32),
            pltpu.SemaphoreType.DMA,
            pltpu.SemaphoreType.DMA,
            pltpu.SemaphoreType.DMA,
        ],
        compiler_params=pltpu.CompilerParams(needs_layout_passes=False),
    )(bm, pk)
    seg = seg_full[:n]

    h, s1, s2 = pl.pallas_call(
        _combine_body,
        out_shape=[
            jax.ShapeDtypeStruct((n, d), jnp.float32),
            jax.ShapeDtypeStruct((1, d), jnp.float32),
            jax.ShapeDtypeStruct((1, d), jnp.float32),
        ],
    )(a_mat, seg)
    out = pl.pallas_call(
        _norm_body,
        out_shape=jax.ShapeDtypeStruct((n, d), jnp.float32),
    )(h, s1, s2, gamma.reshape(1, d), beta.reshape(1, d))
    return out
